# baseline (speedup 1.0000x reference)
import functools

import jax
import jax.numpy as jnp
from jax import lax
from jax.experimental import pallas as pl
from jax.experimental.pallas import tpu as pltpu
from jax.experimental.pallas import tpu_sc as plsc

_X = 1000000
_YC = 32
_YCP = 33
_N = 819200
_LO = -1.0

_NC, _NS, _L = 2, 16, 16
_NW = _NC * _NS
_QW = _N // _NW
_CHUNK = 1024
_NCHUNK = _QW // _CHUNK
_BQ = 128
_NB = _CHUNK // _BQ
_NG = _BQ // _L


def _series_eval(z, cheb_padded):
    mesh = plsc.VectorSubcoreMesh(core_axis_name="c", subcore_axis_name="s")

    @functools.partial(
        pl.kernel,
        out_type=jax.ShapeDtypeStruct((_N,), jnp.float32),
        mesh=mesh,
        compiler_params=pltpu.CompilerParams(
            needs_layout_passes=False, use_tc_tiling_on_sc=False),
        scratch_types=[
            pltpu.VMEM((_CHUNK,), jnp.float32),
            pltpu.VMEM((_NB, _BQ), jnp.int32),
            pltpu.VMEM((_CHUNK,), jnp.float32),
            pltpu.VMEM((_NB, _BQ, _YCP), jnp.float32),
            pltpu.VMEM((_CHUNK,), jnp.float32),
            pltpu.SemaphoreType.DMA,
        ],
    )
    def run(z_hbm, cheb_hbm, out_hbm, z_v, idx_v, y_v, rows_v, out_v, sem):
        wid = lax.axis_index("s") * _NC + lax.axis_index("c")
        base = wid * _QW

        def chunk_body(c, carry):
            off = base + c * _CHUNK
            pltpu.sync_copy(z_hbm.at[pl.ds(off, _CHUNK)], z_v)

            for i in range(_CHUNK // _L):
                t = z_v[pl.ds(i * _L, _L)] - _LO
                xi = (t * 0.5).astype(jnp.int32)
                xi = jnp.minimum(xi, _X - 1)
                y = t - 2.0 * xi.astype(jnp.float32) + _LO
                y = jnp.minimum(jnp.maximum(y, -1.0 + 1e-6), 1.0 - 1e-6)
                idx_v[i // _NG, pl.ds((i % _NG) * _L, _L)] = xi
                y_v[pl.ds(i * _L, _L)] = y

            copies = [
                pltpu.async_copy(cheb_hbm.at[idx_v.at[b]], rows_v.at[b], sem)
                for b in range(_NB)
            ]
            for cp in copies:
                cp.wait()

            for b in range(_NB):
                rows_b = rows_v.at[b]

                def group_body(g, _, b=b, rows_b=rows_b):
                    q0 = b * _BQ + g * _L
                    qidx = lax.iota(jnp.int32, _L) + g * _L
                    y = y_v[pl.ds(q0, _L)]
                    y2 = y + y
                    bk1 = plsc.load_gather(
                        rows_b, [qidx, jnp.full((_L,), _YC - 1, jnp.int32)])
                    bk2 = jnp.zeros((_L,), jnp.float32)
                    for n in range(_YC - 2, 0, -1):
                        a = plsc.load_gather(
                            rows_b, [qidx, jnp.full((_L,), n, jnp.int32)])
                        bk1, bk2 = a + y2 * bk1 - bk2, bk1
                    a0 = plsc.load_gather(
                        rows_b, [qidx, jnp.full((_L,), 0, jnp.int32)])
                    out_v[pl.ds(q0, _L)] = a0 + y * bk1 - bk2
                    return _

                lax.fori_loop(0, _NG, group_body, 0)

            pltpu.sync_copy(out_v, out_hbm.at[pl.ds(off, _CHUNK)])
            return carry

        lax.fori_loop(0, _NCHUNK, chunk_body, 0)

    return run(z, cheb_padded)


def kernel(z, cheb):
    cheb_padded = jnp.pad(cheb, ((0, 0), (0, _YCP - _YC)))
    return _series_eval(z, cheb_padded)


# speedup vs baseline: 1.1418x; 1.1418x over previous
import functools

import jax
import jax.numpy as jnp
from jax import lax
from jax.experimental import pallas as pl
from jax.experimental.pallas import tpu as pltpu
from jax.experimental.pallas import tpu_sc as plsc

_X = 1000000
_YC = 32
_YCP = 40
_N = 819200
_LO = -1.0

_NC, _NS, _L = 2, 16, 16
_NW = _NC * _NS
_QW = _N // _NW
_CHUNK = 1024
_NCHUNK = _QW // _CHUNK
_BQ = 128
_NB = _CHUNK // _BQ
_NG = _BQ // _L

_AW = 512
_AFULL = _X // _AW
_ATAIL = _X - _AFULL * _AW
_AIT = (_AFULL + 1 + _NW - 1) // _NW


def _detile_pad(cheb):
    mesh = plsc.VectorSubcoreMesh(core_axis_name="c", subcore_axis_name="s")

    @functools.partial(
        pl.kernel,
        out_type=jax.ShapeDtypeStruct((_X * _YCP,), jnp.float32),
        mesh=mesh,
        compiler_params=pltpu.CompilerParams(
            needs_layout_passes=False, use_tc_tiling_on_sc=True),
        scratch_types=[
            pltpu.VMEM((_AW, _YC), jnp.float32),
            pltpu.VMEM((_AW * _YCP,), jnp.float32),
        ],
    )
    def detile(cheb_hbm, out_hbm, cin, cout):
        wid = lax.axis_index("s") * _NC + lax.axis_index("c")

        def restride(rows):
            def it_body(it, carry):
                for j in range(16):
                    r = it * 16 + j
                    cout[pl.ds(r * _YCP, _L)] = cin[r, pl.ds(0, _L)]
                    cout[pl.ds(r * _YCP + _L, _L)] = cin[r, pl.ds(_L, _L)]
                return carry
            lax.fori_loop(0, rows // 16, it_body, 0)

        def blk(i, carry):
            b = i * _NW + wid

            @pl.when(b < _AFULL)
            def _():
                r0 = b * _AW
                pltpu.sync_copy(cheb_hbm.at[pl.ds(r0, _AW), :], cin)
                restride(_AW)
                pltpu.sync_copy(cout, out_hbm.at[pl.ds(r0 * _YCP, _AW * _YCP)])

            @pl.when(b == _AFULL)
            def _():
                r0 = _AFULL * _AW
                pltpu.sync_copy(
                    cheb_hbm.at[pl.ds(r0, _ATAIL), :],
                    cin.at[pl.ds(0, _ATAIL), :])
                restride(_ATAIL)
                pltpu.sync_copy(
                    cout.at[pl.ds(0, _ATAIL * _YCP)],
                    out_hbm.at[pl.ds(r0 * _YCP, _ATAIL * _YCP)])

            return carry

        lax.fori_loop(0, _AIT, blk, 0)

    return detile(cheb)


def _series_eval(z, table):
    mesh = plsc.VectorSubcoreMesh(core_axis_name="c", subcore_axis_name="s")

    @functools.partial(
        pl.kernel,
        out_type=jax.ShapeDtypeStruct((_N,), jnp.float32),
        mesh=mesh,
        compiler_params=pltpu.CompilerParams(
            needs_layout_passes=False, use_tc_tiling_on_sc=False),
        scratch_types=[
            pltpu.VMEM((_CHUNK,), jnp.float32),
            pltpu.VMEM((_NB, _BQ), jnp.int32),
            pltpu.VMEM((_CHUNK,), jnp.float32),
            pltpu.VMEM((_NB, _BQ, _YCP), jnp.float32),
            pltpu.VMEM((_CHUNK,), jnp.float32),
            pltpu.SemaphoreType.DMA,
        ],
    )
    def run(z_hbm, cheb_hbm, out_hbm, z_v, idx_v, y_v, rows_v, out_v, sem):
        wid = lax.axis_index("s") * _NC + lax.axis_index("c")
        base = wid * _QW

        def chunk_body(c, carry):
            off = base + c * _CHUNK
            pltpu.sync_copy(z_hbm.at[pl.ds(off, _CHUNK)], z_v)

            for i in range(_CHUNK // _L):
                t = z_v[pl.ds(i * _L, _L)] - _LO
                xi = (t * 0.5).astype(jnp.int32)
                xi = jnp.minimum(xi, _X - 1)
                y = t - 2.0 * xi.astype(jnp.float32) + _LO
                y = jnp.minimum(jnp.maximum(y, -1.0 + 1e-6), 1.0 - 1e-6)
                idx_v[i // _NG, pl.ds((i % _NG) * _L, _L)] = xi
                y_v[pl.ds(i * _L, _L)] = y

            copies = [
                pltpu.async_copy(cheb_hbm.at[idx_v.at[b]], rows_v.at[b], sem)
                for b in range(_NB)
            ]
            for cp in copies:
                cp.wait()

            for b in range(_NB):
                rows_b = rows_v.at[b]

                def group_body(g, _, b=b, rows_b=rows_b):
                    q0 = b * _BQ + g * _L
                    qidx = lax.iota(jnp.int32, _L) + g * _L
                    y = y_v[pl.ds(q0, _L)]
                    y2 = y + y
                    bk1 = plsc.load_gather(
                        rows_b, [qidx, jnp.full((_L,), _YC - 1, jnp.int32)])
                    bk2 = jnp.zeros((_L,), jnp.float32)
                    for n in range(_YC - 2, 0, -1):
                        a = plsc.load_gather(
                            rows_b, [qidx, jnp.full((_L,), n, jnp.int32)])
                        bk1, bk2 = a + y2 * bk1 - bk2, bk1
                    a0 = plsc.load_gather(
                        rows_b, [qidx, jnp.full((_L,), 0, jnp.int32)])
                    out_v[pl.ds(q0, _L)] = a0 + y * bk1 - bk2
                    return _

                lax.fori_loop(0, _NG, group_body, 0)

            pltpu.sync_copy(out_v, out_hbm.at[pl.ds(off, _CHUNK)])
            return carry

        lax.fori_loop(0, _NCHUNK, chunk_body, 0)

    return run(z, table)


def kernel(z, cheb):
    flat = _detile_pad(cheb)
    return _series_eval(z, flat.reshape(_X, _YCP))


# speedup vs baseline: 1.4441x; 1.2648x over previous
import functools

import jax
import jax.numpy as jnp
from jax import lax
from jax.experimental import pallas as pl
from jax.experimental.pallas import tpu as pltpu
from jax.experimental.pallas import tpu_sc as plsc

_X = 1000000
_YC = 32
_YCP = 40
_N = 819200
_LO = -1.0

_NC, _NS, _L = 2, 16, 16
_NW = _NC * _NS
_QW = _N // _NW
_CHUNK = 1024
_NCHUNK = _QW // _CHUNK
_BQ = 128
_NB = _CHUNK // _BQ
_NG = _BQ // _L

_AW = 256
_AFULL = 3904
_APW = _AFULL // _NW
_ATAIL0 = _AFULL * _AW
_AT1 = 256
_AT2 = _X - _ATAIL0 - 2 * _AT1


def _detile_pad(cheb):
    mesh = plsc.VectorSubcoreMesh(core_axis_name="c", subcore_axis_name="s")

    @functools.partial(
        pl.kernel,
        out_type=jax.ShapeDtypeStruct((_X * _YCP,), jnp.float32),
        mesh=mesh,
        compiler_params=pltpu.CompilerParams(
            needs_layout_passes=False, use_tc_tiling_on_sc=True),
        scratch_types=[
            pltpu.VMEM((_AW, _YC), jnp.float32),
            pltpu.VMEM((_AW, _YC), jnp.float32),
            pltpu.VMEM((_AW * _YCP,), jnp.float32),
            pltpu.VMEM((_AW * _YCP,), jnp.float32),
            pltpu.SemaphoreType.DMA,
            pltpu.SemaphoreType.DMA,
            pltpu.SemaphoreType.DMA,
            pltpu.SemaphoreType.DMA,
        ],
    )
    def detile(cheb_hbm, out_hbm, cin0, cin1, cout0, cout1,
               semr0, semr1, semw0, semw1):
        wid = lax.axis_index("s") * _NC + lax.axis_index("c")
        cins, couts = (cin0, cin1), (cout0, cout1)
        semrs, semws = (semr0, semr1), (semw0, semw1)

        def row0(i):
            return (i * _NW + wid) * _AW

        def read(i, u, start):
            mk = pltpu.async_copy if start else pltpu.make_async_copy
            return mk(cheb_hbm.at[pl.ds(row0(i), _AW), :], cins[u], semrs[u])

        def write(i, u, start):
            mk = pltpu.async_copy if start else pltpu.make_async_copy
            return mk(
                couts[u],
                out_hbm.at[pl.ds(row0(i) * _YCP, _AW * _YCP)], semws[u])

        def restride(cin, cout, rows):
            def it_body(it, carry):
                for j in range(16):
                    r = it * 16 + j
                    cout[pl.ds(r * _YCP, _L)] = cin[r, pl.ds(0, _L)]
                    cout[pl.ds(r * _YCP + _L, _L)] = cin[r, pl.ds(_L, _L)]
                return carry
            lax.fori_loop(0, rows // 16, it_body, 0)

        def step(i, u):
            read(i, u, start=False).wait()

            @pl.when(i + 2 < _APW)
            def _():
                read(i + 2, u, start=True)

            @pl.when(i >= 2)
            def _():
                write(i - 2, u, start=False).wait()

            restride(cins[u], couts[u], _AW)
            write(i, u, start=True)

        read(0, 0, start=True)
        read(1, 1, start=True)

        def pair(j, carry):
            step(2 * j, 0)
            step(2 * j + 1, 1)
            return carry

        lax.fori_loop(0, _APW // 2, pair, 0)
        write(_APW - 2, 0, start=False).wait()
        write(_APW - 1, 1, start=False).wait()

        for w in (0, 1):
            @pl.when(wid == w)
            def _(w=w):
                r0 = _ATAIL0 + w * _AT1
                pltpu.sync_copy(cheb_hbm.at[pl.ds(r0, _AT1), :], cin0)
                restride(cin0, cout0, _AT1)
                pltpu.sync_copy(
                    cout0, out_hbm.at[pl.ds(r0 * _YCP, _AT1 * _YCP)])

        @pl.when(wid == 2)
        def _():
            r0 = _ATAIL0 + 2 * _AT1
            pltpu.sync_copy(
                cheb_hbm.at[pl.ds(r0, _AT2), :],
                cin1.at[pl.ds(0, _AT2), :])
            restride(cin1, cout1, _AT2)
            pltpu.sync_copy(
                cout1.at[pl.ds(0, _AT2 * _YCP)],
                out_hbm.at[pl.ds(r0 * _YCP, _AT2 * _YCP)])

    return detile(cheb)


def _series_eval(z, table):
    mesh = plsc.VectorSubcoreMesh(core_axis_name="c", subcore_axis_name="s")

    @functools.partial(
        pl.kernel,
        out_type=jax.ShapeDtypeStruct((_N,), jnp.float32),
        mesh=mesh,
        compiler_params=pltpu.CompilerParams(
            needs_layout_passes=False, use_tc_tiling_on_sc=False),
        scratch_types=[
            pltpu.VMEM((_CHUNK,), jnp.float32),
            pltpu.VMEM((_NB, _BQ), jnp.int32),
            pltpu.VMEM((_CHUNK,), jnp.float32),
            pltpu.VMEM((_NB, _BQ, _YCP), jnp.float32),
            pltpu.VMEM((_CHUNK,), jnp.float32),
            pltpu.SemaphoreType.DMA,
        ],
    )
    def run(z_hbm, cheb_hbm, out_hbm, z_v, idx_v, y_v, rows_v, out_v, sem):
        wid = lax.axis_index("s") * _NC + lax.axis_index("c")
        base = wid * _QW

        def chunk_body(c, carry):
            off = base + c * _CHUNK
            pltpu.sync_copy(z_hbm.at[pl.ds(off, _CHUNK)], z_v)

            for i in range(_CHUNK // _L):
                t = z_v[pl.ds(i * _L, _L)] - _LO
                xi = (t * 0.5).astype(jnp.int32)
                xi = jnp.minimum(xi, _X - 1)
                y = t - 2.0 * xi.astype(jnp.float32) + _LO
                y = jnp.minimum(jnp.maximum(y, -1.0 + 1e-6), 1.0 - 1e-6)
                idx_v[i // _NG, pl.ds((i % _NG) * _L, _L)] = xi
                y_v[pl.ds(i * _L, _L)] = y

            copies = [
                pltpu.async_copy(cheb_hbm.at[idx_v.at[b]], rows_v.at[b], sem)
                for b in range(_NB)
            ]
            for cp in copies:
                cp.wait()

            for b in range(_NB):
                rows_b = rows_v.at[b]

                def group_body(g, _, b=b, rows_b=rows_b):
                    q0 = b * _BQ + g * _L
                    qidx = lax.iota(jnp.int32, _L) + g * _L
                    y = y_v[pl.ds(q0, _L)]
                    y2 = y + y
                    bk1 = plsc.load_gather(
                        rows_b, [qidx, jnp.full((_L,), _YC - 1, jnp.int32)])
                    bk2 = jnp.zeros((_L,), jnp.float32)
                    for n in range(_YC - 2, 0, -1):
                        a = plsc.load_gather(
                            rows_b, [qidx, jnp.full((_L,), n, jnp.int32)])
                        bk1, bk2 = a + y2 * bk1 - bk2, bk1
                    a0 = plsc.load_gather(
                        rows_b, [qidx, jnp.full((_L,), 0, jnp.int32)])
                    out_v[pl.ds(q0, _L)] = a0 + y * bk1 - bk2
                    return _

                lax.fori_loop(0, _NG, group_body, 0)

            pltpu.sync_copy(out_v, out_hbm.at[pl.ds(off, _CHUNK)])
            return carry

        lax.fori_loop(0, _NCHUNK, chunk_body, 0)

    return run(z, table)


def kernel(z, cheb):
    flat = _detile_pad(cheb)
    return _series_eval(z, flat.reshape(_X, _YCP))
